# Initial kernel scaffold; baseline (speedup 1.0000x reference)
import jax
import jax.numpy as jnp
from jax.experimental import pallas as pl


def kernel(x, edge_index, W_gcn, b_gcn, W1, b1, W2, b2):
    raise NotImplementedError("write your pallas kernel here")


# speedup vs baseline: 36.0136x; 36.0136x over previous
import functools

import jax
import jax.numpy as jnp
from jax import lax
from jax.experimental import pallas as pl
from jax.experimental.pallas import tpu as pltpu
from jax.experimental.pallas import tpu_sc as plsc

N_NODES = 10000
D = 128
DO = 64
NC, NS, LANES = 2, 16, 16
NW = NC * NS
EB = 128
C = 16
R = 10240
RS = R // NS
ROWB = 400

_mesh = plsc.VectorSubcoreMesh(core_axis_name="c", subcore_axis_name="s")

_CHUNKS = [(q * EB, EB) for q in range(RS // EB)]
if RS % EB:
    _CHUNKS.append((RS - RS % EB, RS % EB))
_ZROWS = 64
_DCHUNKS = [(q * _ZROWS, _ZROWS) for q in range(RS // _ZROWS)]


@functools.cache
def _make_deg_kernel(nb):
    @functools.partial(
        pl.kernel,
        mesh=_mesh,
        out_type=jax.ShapeDtypeStruct((NC, R, LANES), jnp.float32),
        scratch_types=[
            pltpu.VMEM((nb, EB), jnp.int32),
            pltpu.VMEM((EB, LANES), jnp.float32),
            pltpu.VMEM((_ZROWS, LANES), jnp.float32),
            pltpu.VMEM_SHARED((R, LANES), jnp.float32),
        ],
    )
    def deg_kernel(dst_hbm, out_hbm, idx_v, ones_v, zeros_v, deg_sh):
        c = lax.axis_index("c")
        s = lax.axis_index("s")
        w = c * NS + s

        def _fill(i, carry):
            ones_v[i, :] = jnp.ones((LANES,), jnp.float32)

            @pl.when(i < _ZROWS)
            def _():
                zeros_v[i, :] = jnp.zeros((LANES,), jnp.float32)

            return carry

        lax.fori_loop(0, EB, _fill, 0)

        base = s * RS
        for off, ln in _DCHUNKS:
            pltpu.sync_copy(zeros_v.at[pl.ds(0, ln)],
                            deg_sh.at[pl.ds(base + off, ln)])
        pltpu.sync_copy(dst_hbm.at[w], idx_v)
        plsc.subcore_barrier()

        def _acc(j, carry):
            pltpu.sync_copy(ones_v, deg_sh.at[idx_v.at[j]], add=True)
            return carry

        lax.fori_loop(0, nb, _acc, 0)
        plsc.subcore_barrier()
        pltpu.sync_copy(deg_sh.at[pl.ds(base, RS)],
                        out_hbm.at[c, pl.ds(base, RS)])

    return deg_kernel


@functools.cache
def _make_agg_kernel(nb):
    nch = nb // C

    @functools.partial(
        pl.kernel,
        mesh=_mesh,
        out_type=jax.ShapeDtypeStruct((NC, R, D), jnp.float32),
        scratch_types=[
            pltpu.VMEM((2, C, EB), jnp.int32),
            pltpu.VMEM((2, C, EB), jnp.int32),
            pltpu.VMEM((EB, D), jnp.float32),
            pltpu.VMEM((EB, D), jnp.float32),
            pltpu.VMEM_SHARED((R, D), jnp.float32),
            pltpu.SemaphoreType.DMA,
            pltpu.SemaphoreType.DMA,
            pltpu.SemaphoreType.DMA,
        ],
    )
    def agg_kernel(src_hbm, dst_hbm, h2_hbm, out_hbm,
                   srcc, dstc, bufa, bufb, agg_sh, sema, semb, semi):
        c = lax.axis_index("c")
        s = lax.axis_index("s")
        w = c * NS + s

        def _zero(i, carry):
            for k in range(D // LANES):
                bufa[i, pl.ds(k * LANES, LANES)] = jnp.zeros((LANES,),
                                                             jnp.float32)
            return carry

        lax.fori_loop(0, EB, _zero, 0)
        base = s * RS
        for off, ln in _CHUNKS:
            pltpu.sync_copy(bufa.at[pl.ds(0, ln)],
                            agg_sh.at[pl.ds(base + off, ln)])
        pltpu.sync_copy(src_hbm.at[w, pl.ds(0, C)], srcc.at[0])
        pltpu.sync_copy(dst_hbm.at[w, pl.ds(0, C)], dstc.at[0])
        plsc.subcore_barrier()

        def _chunk(g, carry):
            p = lax.rem(g, 2)
            pn = lax.rem(g + 1, 2)

            @pl.when(g + 1 < nch)
            def _():
                pltpu.async_copy(src_hbm.at[w, pl.ds((g + 1) * C, C)],
                                 srcc.at[pn], semi)
                pltpu.async_copy(dst_hbm.at[w, pl.ds((g + 1) * C, C)],
                                 dstc.at[pn], semi)

            pltpu.async_copy(h2_hbm.at[srcc.at[p, 0]], bufa, sema)

            def _pair(t, carry2):
                j = t * 2
                pltpu.async_copy(h2_hbm.at[srcc.at[p, j + 1]], bufb, semb)
                pltpu.make_async_copy(h2_hbm.at[srcc.at[p, j]], bufa,
                                      sema).wait()
                pltpu.sync_copy(bufa, agg_sh.at[dstc.at[p, j]], add=True)

                @pl.when(j + 2 < C)
                def _():
                    pltpu.async_copy(h2_hbm.at[srcc.at[p, j + 2]], bufa, sema)

                pltpu.make_async_copy(h2_hbm.at[srcc.at[p, j + 1]], bufb,
                                      semb).wait()
                pltpu.sync_copy(bufb, agg_sh.at[dstc.at[p, j + 1]], add=True)
                return carry2

            lax.fori_loop(0, C // 2, _pair, 0)

            @pl.when(g + 1 < nch)
            def _():
                pltpu.make_async_copy(src_hbm.at[w, pl.ds((g + 1) * C, C)],
                                      srcc.at[pn], semi).wait()
                pltpu.make_async_copy(dst_hbm.at[w, pl.ds((g + 1) * C, C)],
                                      dstc.at[pn], semi).wait()

            return carry

        lax.fori_loop(0, nch, _chunk, 0)
        plsc.subcore_barrier()
        pltpu.sync_copy(agg_sh.at[pl.ds(base, RS)],
                        out_hbm.at[c, pl.ds(base, RS)])

    return agg_kernel


def _h2_body(x_ref, w_ref, degp_ref, o_ref):
    dg = degp_ref[...]
    deg = dg[0, :, 0:1] + dg[1, :, 0:1] + 1.0
    dinv = lax.rsqrt(deg)
    h = jnp.dot(x_ref[...], w_ref[...], preferred_element_type=jnp.float32)
    o_ref[...] = h * dinv


_h2_call = pl.pallas_call(
    _h2_body,
    grid=(N_NODES // ROWB,),
    in_specs=[
        pl.BlockSpec((ROWB, D), lambda i: (i, 0)),
        pl.BlockSpec((D, D), lambda i: (0, 0)),
        pl.BlockSpec((NC, ROWB, LANES), lambda i: (0, i, 0)),
    ],
    out_specs=pl.BlockSpec((ROWB, D), lambda i: (i, 0)),
    out_shape=jax.ShapeDtypeStruct((N_NODES, D), jnp.float32),
)


def _mlp_body(aggp_ref, degp_ref, h2_ref, bg_ref, w1_ref, b1_ref, w2_ref,
              b2_ref, o_ref):
    p = aggp_ref[...]
    dg = degp_ref[...]
    deg = dg[0, :, 0:1] + dg[1, :, 0:1] + 1.0
    dinv = lax.rsqrt(deg)
    t = (p[0] + p[1] + h2_ref[...]) * dinv
    a = jnp.maximum(t + bg_ref[...], 0.0)
    m = jnp.maximum(
        jnp.dot(a, w1_ref[...], preferred_element_type=jnp.float32)
        + b1_ref[...], 0.0)
    o = jnp.dot(m, w2_ref[...], preferred_element_type=jnp.float32) + b2_ref[...]
    mx = jnp.max(o, axis=1, keepdims=True)
    lse = mx + jnp.log(jnp.sum(jnp.exp(o - mx), axis=1, keepdims=True))
    o_ref[...] = o - lse


_mlp_call = pl.pallas_call(
    _mlp_body,
    grid=(N_NODES // ROWB,),
    in_specs=[
        pl.BlockSpec((NC, ROWB, D), lambda i: (0, i, 0)),
        pl.BlockSpec((NC, ROWB, LANES), lambda i: (0, i, 0)),
        pl.BlockSpec((ROWB, D), lambda i: (i, 0)),
        pl.BlockSpec((1, D), lambda i: (0, 0)),
        pl.BlockSpec((D, D), lambda i: (0, 0)),
        pl.BlockSpec((1, D), lambda i: (0, 0)),
        pl.BlockSpec((D, DO), lambda i: (0, 0)),
        pl.BlockSpec((1, DO), lambda i: (0, 0)),
    ],
    out_specs=pl.BlockSpec((ROWB, DO), lambda i: (i, 0)),
    out_shape=jax.ShapeDtypeStruct((N_NODES, DO), jnp.float32),
)


def kernel(x, edge_index, W_gcn, b_gcn, W1, b1, W2, b2):
    e = edge_index.shape[1]
    nb = -(-e // (NW * EB))
    nb += (-nb) % C
    padn = NW * nb * EB - e

    ei = edge_index.astype(jnp.int32)
    pidx = jnp.arange(padn, dtype=jnp.int32)
    pad_src = (pidx * 131) % N_NODES
    pad_dst = N_NODES + pidx % (R - N_NODES)
    src = jnp.concatenate([ei[0], pad_src]).reshape(NW, nb, EB)
    dst = jnp.concatenate([ei[1], pad_dst]).reshape(NW, nb, EB)

    degp = _make_deg_kernel(nb)(dst)
    h2 = _h2_call(x, W_gcn, degp)
    aggp = _make_agg_kernel(nb)(src, dst, h2)
    return _mlp_call(aggp, degp, h2, b_gcn.reshape(1, D), W1,
                     b1.reshape(1, D), W2, b2.reshape(1, DO))
